# Initial kernel scaffold; baseline (speedup 1.0000x reference)
import jax
import jax.numpy as jnp
from jax.experimental import pallas as pl


def kernel(x, W0, W1, W2, W3, W4, W5, W6, W7, W8):
    raise NotImplementedError("write your pallas kernel here")


# speedup vs baseline: 10.6867x; 10.6867x over previous
import jax
import jax.numpy as jnp
import numpy as np
from jax.experimental import pallas as pl

_DIMS = [119, 5, 12, 12, 10, 6, 6, 2, 2]
_EMB = 128
_TPAD = 256
_OFFS = np.concatenate([[0], np.cumsum(_DIMS)[:-1]]).astype(np.int32)
_BLOCK = 2000


def _body(x_ref, t_ref, o_ref):
    x = x_ref[...]
    iota = jax.lax.broadcasted_iota(jnp.int32, (x.shape[0], _TPAD), 1)
    m = jnp.zeros((x.shape[0], _TPAD), jnp.float32)
    for i in range(x.shape[1]):
        m = m + (iota == x[:, i : i + 1] + int(_OFFS[i])).astype(jnp.float32)
    o_ref[...] = jnp.dot(m, t_ref[...], preferred_element_type=jnp.float32)


def kernel(x, W0, W1, W2, W3, W4, W5, W6, W7, W8):
    n = x.shape[0]
    table = jnp.zeros((_TPAD, _EMB), jnp.float32)
    row = 0
    for w in (W0, W1, W2, W3, W4, W5, W6, W7, W8):
        table = jax.lax.dynamic_update_slice(table, w, (row, 0))
        row += w.shape[0]
    grid = n // _BLOCK
    return pl.pallas_call(
        _body,
        grid=(grid,),
        in_specs=[
            pl.BlockSpec((_BLOCK, x.shape[1]), lambda i: (i, 0)),
            pl.BlockSpec((_TPAD, _EMB), lambda i: (0, 0)),
        ],
        out_specs=pl.BlockSpec((_BLOCK, _EMB), lambda i: (i, 0)),
        out_shape=jax.ShapeDtypeStruct((n, _EMB), jnp.float32),
    )(x, table)
